# baseline (speedup 1.0000x reference)
import functools

import jax
import jax.numpy as jnp
from jax import lax
from jax.experimental import pallas as pl
from jax.experimental.pallas import tpu as pltpu
from jax.experimental.pallas import tpu_sc as plsc

_TEMP = 0.1
_B, _IN, _HID, _EMB, _QSZ = 1024, 512, 1024, 256, 65536
_B2 = 2 * _B
_QBLK = 1024
_NBLK = _QSZ // _QBLK
_CHUNK = 256
_PREC = lax.Precision.DEFAULT


def _rownorm(a):
    n = jnp.sqrt(jnp.sum(a * a, axis=1, keepdims=True))
    return a / jnp.maximum(n, 1e-12)


def _front(x, n1, n2, we1, be1, we2, be2, wp, bp, wq, bq):
    aug = jnp.concatenate([x + n1, x + n2], axis=0)
    h = jnp.maximum(
        jnp.dot(aug, we1, preferred_element_type=jnp.float32,
                precision=_PREC) + be1, 0.0)
    f = jnp.maximum(
        jnp.dot(h, we2, preferred_element_type=jnp.float32,
                precision=_PREC) + be2, 0.0)
    proj = jnp.dot(f, wp, preferred_element_type=jnp.float32,
                   precision=_PREC) + bp
    pred = jnp.dot(proj, wq, preferred_element_type=jnp.float32,
                   precision=_PREC) + bq
    return proj, pred


def _extract(s, base):
    bm = jnp.max(s, axis=1, keepdims=True)
    cols = lax.broadcasted_iota(jnp.int32, (1, _CHUNK), 1).astype(
        jnp.bfloat16)
    parts = []
    for c in range(_QBLK // _CHUNK):
        sc = s[:, c * _CHUNK:(c + 1) * _CHUNK]
        cand = jnp.where(sc >= bm, cols, jnp.bfloat16(1024.0))
        parts.append(jnp.min(cand, axis=1, keepdims=True).astype(jnp.float32)
                     + c * _CHUNK)
    bi = jnp.minimum(jnp.minimum(parts[0], parts[1]),
                     jnp.minimum(parts[2], parts[3])).astype(jnp.int32) + base
    return bm, bi


def _sims_body(x_ref, n1_ref, n2_ref, we1_ref, be1_ref, we2_ref, be2_ref,
               wp_ref, bp_ref, wq_ref, bq_ref, q0_ref, q1_ref, q2_ref,
               q3_ref, idx_ref, prd_ref, pbf_ref, cbuf_ref, dbuf_ref,
               rmax_ref, ridx_ref):
    j = pl.program_id(0)

    @pl.when(j == 0)
    def _do_front():
        proj, pred = _front(x_ref[...], n1_ref[...], n2_ref[...],
                            we1_ref[...], be1_ref[...], we2_ref[...],
                            be2_ref[...], wp_ref[...], bp_ref[...],
                            wq_ref[...], bq_ref[...])
        pbf_ref[...] = proj.astype(jnp.bfloat16)
        prd_ref[...] = _rownorm(pred)

    p_bf = pbf_ref[...]
    dn = (((1,), (1,)), ((), ()))

    def blockdot(q_ref):
        return lax.dot_general(p_bf, q_ref[...].astype(jnp.bfloat16), dn,
                               preferred_element_type=jnp.float32
                               ).astype(jnp.bfloat16)

    bm_c, bi_c = _extract(cbuf_ref[...], (4 * j - 2) * _QBLK)
    bm_d, bi_d = _extract(dbuf_ref[...], (4 * j - 1) * _QBLK)
    cbuf_ref[...] = blockdot(q2_ref)
    dbuf_ref[...] = blockdot(q3_ref)
    bm_a, bi_a = _extract(blockdot(q0_ref), 4 * j * _QBLK)
    bm_b, bi_b = _extract(blockdot(q1_ref), (4 * j + 1) * _QBLK)

    rmax = jnp.where(j == 0, jnp.bfloat16(-jnp.inf), rmax_ref[...])
    ridx = ridx_ref[...]
    for bm, bi, ok in ((bm_c, bi_c, j > 0), (bm_d, bi_d, j > 0),
                       (bm_a, bi_a, 4 * j < _NBLK),
                       (bm_b, bi_b, 4 * j + 1 < _NBLK)):
        upd = jnp.logical_and(bm > rmax, ok)
        ridx = jnp.where(upd, bi, ridx)
        rmax = jnp.where(upd, bm, rmax)
    ridx_ref[...] = ridx
    rmax_ref[...] = rmax

    @pl.when(j == _NBLK // 4)
    def _emit():
        idx_ref[...] = ridx_ref[...]


def _loss_body(nnf_ref, prd_ref, out_ref):
    nnf = _rownorm(nnf_ref[...])
    prd = prd_ref[...]
    nn1, nn2 = nnf[:_B], nnf[_B:]
    prd1, prd2 = prd[:_B], prd[_B:]

    def half(prd_h, nn_h):
        logits = lax.dot_general(
            prd_h, nn_h, (((1,), (1,)), ((), ())),
            preferred_element_type=jnp.float32, precision=_PREC) / _TEMP
        m = jnp.max(logits, axis=1, keepdims=True)
        lse = m[:, 0] + jnp.log(jnp.sum(jnp.exp(logits - m), axis=1))
        diag = jnp.sum(prd_h * nn_h, axis=1) / _TEMP
        return jnp.mean(lse - diag)

    loss1 = half(prd2, nn1)
    loss2 = half(prd1, nn2)
    out_ref[0, 0] = 0.5 * (loss1 + loss2)


_NC, _NS = 2, 16
_NW = _NC * _NS
_BPW = _B2 // _NW


@functools.lru_cache(maxsize=1)
def _gather_fn():
    @functools.partial(
        pl.kernel,
        mesh=plsc.VectorSubcoreMesh(core_axis_name="c", subcore_axis_name="s"),
        out_type=jax.ShapeDtypeStruct((_B2, _EMB), jnp.float32),
        scratch_types=[
            pltpu.VMEM((_BPW,), jnp.int32),
            pltpu.VMEM((_BPW, _EMB), jnp.float32),
            pltpu.SemaphoreType.DMA,
        ],
    )
    def _gather(table_hbm, idx_hbm, out_hbm, idx_v, rows_v, sem):
        wid = lax.axis_index("s") * _NC + lax.axis_index("c")
        base = wid * _BPW
        pltpu.sync_copy(idx_hbm.at[pl.ds(base, _BPW)], idx_v)
        pltpu.async_copy(table_hbm.at[idx_v], rows_v, sem).wait()
        pltpu.sync_copy(rows_v, out_hbm.at[pl.ds(base, _BPW)])

    return _gather


def kernel(x, noise1, noise2, feature_queue, W_e1, b_e1, W_e2, b_e2,
           W_p, b_p, W_q, b_q):
    f32 = jnp.float32
    cmap = lambda j: (0, 0)
    nn_idx, prdnorm = pl.pallas_call(
        _sims_body,
        grid=(_NBLK // 4 + 1,),
        in_specs=[
            pl.BlockSpec((_B, _IN), cmap),
            pl.BlockSpec((_B, _IN), cmap),
            pl.BlockSpec((_B, _IN), cmap),
            pl.BlockSpec((_IN, _HID), cmap),
            pl.BlockSpec((1, _HID), cmap),
            pl.BlockSpec((_HID, _EMB), cmap),
            pl.BlockSpec((1, _EMB), cmap),
            pl.BlockSpec((_EMB, _EMB), cmap),
            pl.BlockSpec((1, _EMB), cmap),
            pl.BlockSpec((_EMB, _EMB), cmap),
            pl.BlockSpec((1, _EMB), cmap),
            pl.BlockSpec((_QBLK, _EMB),
                         lambda j: (jnp.minimum(4 * j, _NBLK - 1), 0)),
            pl.BlockSpec((_QBLK, _EMB),
                         lambda j: (jnp.minimum(4 * j + 1, _NBLK - 1), 0)),
            pl.BlockSpec((_QBLK, _EMB),
                         lambda j: (jnp.minimum(4 * j + 2, _NBLK - 1), 0)),
            pl.BlockSpec((_QBLK, _EMB),
                         lambda j: (jnp.minimum(4 * j + 3, _NBLK - 1), 0)),
        ],
        out_specs=(pl.BlockSpec((_B2, 1), cmap),
                   pl.BlockSpec((_B2, _EMB), cmap)),
        out_shape=(jax.ShapeDtypeStruct((_B2, 1), jnp.int32),
                   jax.ShapeDtypeStruct((_B2, _EMB), f32)),
        scratch_shapes=[
            pltpu.VMEM((_B2, _EMB), jnp.bfloat16),
            pltpu.VMEM((_B2, _QBLK), jnp.bfloat16),
            pltpu.VMEM((_B2, _QBLK), jnp.bfloat16),
            pltpu.VMEM((_B2, 1), jnp.bfloat16),
            pltpu.VMEM((_B2, 1), jnp.int32),
        ],
    )(x, noise1, noise2, W_e1, b_e1.reshape(1, _HID), W_e2,
      b_e2.reshape(1, _EMB), W_p, b_p.reshape(1, _EMB), W_q,
      b_q.reshape(1, _EMB), feature_queue, feature_queue, feature_queue,
      feature_queue)

    nnf = _gather_fn()(feature_queue, nn_idx.reshape(_B2))

    out = pl.pallas_call(
        _loss_body,
        out_specs=pl.BlockSpec(memory_space=pltpu.SMEM),
        out_shape=jax.ShapeDtypeStruct((1, 1), f32),
    )(nnf, prdnorm)
    return out[0, 0]


# speedup vs baseline: 1.0957x; 1.0957x over previous
import functools

import jax
import jax.numpy as jnp
from jax import lax
from jax.experimental import pallas as pl
from jax.experimental.pallas import tpu as pltpu
from jax.experimental.pallas import tpu_sc as plsc

_TEMP = 0.1
_B, _IN, _HID, _EMB, _QSZ = 1024, 512, 1024, 256, 65536
_B2 = 2 * _B
_QBLK = 2048
_NBLK = _QSZ // _QBLK
_CHUNK = 256
_PREC = lax.Precision.DEFAULT


def _rownorm(a):
    n = jnp.sqrt(jnp.sum(a * a, axis=1, keepdims=True))
    return a / jnp.maximum(n, 1e-12)


def _front(x, n1, n2, we1, be1, we2, be2, wp, bp, wq, bq):
    aug = jnp.concatenate([x + n1, x + n2], axis=0)
    h = jnp.maximum(
        jnp.dot(aug, we1, preferred_element_type=jnp.float32,
                precision=_PREC) + be1, 0.0)
    f = jnp.maximum(
        jnp.dot(h, we2, preferred_element_type=jnp.float32,
                precision=_PREC) + be2, 0.0)
    proj = jnp.dot(f, wp, preferred_element_type=jnp.float32,
                   precision=_PREC) + bp
    pred = jnp.dot(proj, wq, preferred_element_type=jnp.float32,
                   precision=_PREC) + bq
    return proj, pred


def _extract(s, base):
    bm = jnp.max(s, axis=1, keepdims=True)
    cols = lax.broadcasted_iota(jnp.int32, (1, _CHUNK), 1).astype(
        jnp.bfloat16)
    parts = []
    for c in range(_QBLK // _CHUNK):
        sc = s[:, c * _CHUNK:(c + 1) * _CHUNK]
        cand = jnp.where(sc >= bm, cols, jnp.bfloat16(float(_QBLK)))
        parts.append(jnp.min(cand, axis=1, keepdims=True).astype(jnp.float32)
                     + c * _CHUNK)
    while len(parts) > 1:
        parts = [jnp.minimum(parts[k], parts[k + 1])
                 for k in range(0, len(parts), 2)]
    bi = parts[0].astype(jnp.int32) + base
    return bm, bi


def _sims_body(x_ref, n1_ref, n2_ref, we1_ref, be1_ref, we2_ref, be2_ref,
               wp_ref, bp_ref, wq_ref, bq_ref, q0_ref, q1_ref,
               idx_ref, prd_ref, pbf_ref, sbuf_ref, rmax_ref, ridx_ref):
    j = pl.program_id(0)

    @pl.when(j == 0)
    def _do_front():
        proj, pred = _front(x_ref[...], n1_ref[...], n2_ref[...],
                            we1_ref[...], be1_ref[...], we2_ref[...],
                            be2_ref[...], wp_ref[...], bp_ref[...],
                            wq_ref[...], bq_ref[...])
        pbf_ref[...] = proj.astype(jnp.bfloat16)
        prd_ref[...] = _rownorm(pred)

    p_bf = pbf_ref[...]
    dn = (((1,), (1,)), ((), ()))
    s_even = lax.dot_general(p_bf, q0_ref[...].astype(jnp.bfloat16), dn,
                             preferred_element_type=jnp.float32
                             ).astype(jnp.bfloat16)
    s_odd_prev = sbuf_ref[...]
    bm_o, bi_o = _extract(s_odd_prev, (2 * j - 1) * _QBLK)
    sbuf_ref[...] = lax.dot_general(p_bf, q1_ref[...].astype(jnp.bfloat16),
                                    dn, preferred_element_type=jnp.float32
                                    ).astype(jnp.bfloat16)
    bm_e, bi_e = _extract(s_even, 2 * j * _QBLK)

    rmax = jnp.where(j == 0, jnp.bfloat16(-jnp.inf), rmax_ref[...])
    ridx = ridx_ref[...]
    ok_o = jnp.logical_and(bm_o > rmax, j > 0)
    ridx = jnp.where(ok_o, bi_o, ridx)
    rmax = jnp.where(ok_o, bm_o, rmax)
    ok_e = jnp.logical_and(bm_e > rmax, 2 * j < _NBLK)
    ridx = jnp.where(ok_e, bi_e, ridx)
    rmax = jnp.where(ok_e, bm_e, rmax)
    ridx_ref[...] = ridx
    rmax_ref[...] = rmax
    idx_ref[...] = ridx


def _loss_body(nnf_ref, prd_ref, out_ref):
    nnf = _rownorm(nnf_ref[...])
    prd = prd_ref[...]
    nn1, nn2 = nnf[:_B], nnf[_B:]
    prd1, prd2 = prd[:_B], prd[_B:]

    def half(prd_h, nn_h):
        logits = lax.dot_general(
            prd_h, nn_h, (((1,), (1,)), ((), ())),
            preferred_element_type=jnp.float32, precision=_PREC) / _TEMP
        m = jnp.max(logits, axis=1, keepdims=True)
        lse = m[:, 0] + jnp.log(jnp.sum(jnp.exp(logits - m), axis=1))
        diag = jnp.sum(prd_h * nn_h, axis=1) / _TEMP
        return jnp.mean(lse - diag)

    loss1 = half(prd2, nn1)
    loss2 = half(prd1, nn2)
    out_ref[0, 0] = 0.5 * (loss1 + loss2)


_NC, _NS = 2, 16
_NW = _NC * _NS
_BPW = _B2 // _NW


@functools.lru_cache(maxsize=1)
def _gather_fn():
    @functools.partial(
        pl.kernel,
        mesh=plsc.VectorSubcoreMesh(core_axis_name="c", subcore_axis_name="s"),
        out_type=jax.ShapeDtypeStruct((_B2, _EMB), jnp.float32),
        scratch_types=[
            pltpu.VMEM((_BPW,), jnp.int32),
            pltpu.VMEM((_BPW, _EMB), jnp.float32),
            pltpu.SemaphoreType.DMA,
        ],
    )
    def _gather(table_hbm, idx_hbm, out_hbm, idx_v, rows_v, sem):
        wid = lax.axis_index("s") * _NC + lax.axis_index("c")
        base = wid * _BPW
        pltpu.sync_copy(idx_hbm.at[pl.ds(base, _BPW)], idx_v)
        pltpu.async_copy(table_hbm.at[idx_v], rows_v, sem).wait()
        pltpu.sync_copy(rows_v, out_hbm.at[pl.ds(base, _BPW)])

    return _gather


def kernel(x, noise1, noise2, feature_queue, W_e1, b_e1, W_e2, b_e2,
           W_p, b_p, W_q, b_q):
    f32 = jnp.float32
    cmap = lambda j: (0, 0)
    nn_idx, prdnorm = pl.pallas_call(
        _sims_body,
        grid=(_NBLK // 2 + 1,),
        in_specs=[
            pl.BlockSpec((_B, _IN), cmap),
            pl.BlockSpec((_B, _IN), cmap),
            pl.BlockSpec((_B, _IN), cmap),
            pl.BlockSpec((_IN, _HID), cmap),
            pl.BlockSpec((1, _HID), cmap),
            pl.BlockSpec((_HID, _EMB), cmap),
            pl.BlockSpec((1, _EMB), cmap),
            pl.BlockSpec((_EMB, _EMB), cmap),
            pl.BlockSpec((1, _EMB), cmap),
            pl.BlockSpec((_EMB, _EMB), cmap),
            pl.BlockSpec((1, _EMB), cmap),
            pl.BlockSpec((_QBLK, _EMB),
                         lambda j: (jnp.minimum(2 * j, _NBLK - 1), 0)),
            pl.BlockSpec((_QBLK, _EMB),
                         lambda j: (jnp.minimum(2 * j + 1, _NBLK - 1), 0)),
        ],
        out_specs=(pl.BlockSpec((_B2, 1), cmap),
                   pl.BlockSpec((_B2, _EMB), cmap)),
        out_shape=(jax.ShapeDtypeStruct((_B2, 1), jnp.int32),
                   jax.ShapeDtypeStruct((_B2, _EMB), f32)),
        scratch_shapes=[
            pltpu.VMEM((_B2, _EMB), jnp.bfloat16),
            pltpu.VMEM((_B2, _QBLK), jnp.bfloat16),
            pltpu.VMEM((_B2, 1), jnp.bfloat16),
            pltpu.VMEM((_B2, 1), jnp.int32),
        ],
    )(x, noise1, noise2, W_e1, b_e1.reshape(1, _HID), W_e2,
      b_e2.reshape(1, _EMB), W_p, b_p.reshape(1, _EMB), W_q,
      b_q.reshape(1, _EMB), feature_queue, feature_queue)

    nnf = _gather_fn()(feature_queue, nn_idx.reshape(_B2))

    out = pl.pallas_call(
        _loss_body,
        out_specs=pl.BlockSpec(memory_space=pltpu.SMEM),
        out_shape=jax.ShapeDtypeStruct((1, 1), f32),
    )(nnf, prdnorm)
    return out[0, 0]
